# Initial kernel scaffold; baseline (speedup 1.0000x reference)
import jax
import jax.numpy as jnp
from jax.experimental import pallas as pl


def kernel(x_entity, x_item, W_fe_e, b_fe_e, W_fe_i, b_fe_i, W_ne_e, b_ne_e, W_ne_i, b_ne_i, W_time, b_time, emb_table, W_self_e, W_neigh_e, W_self_i, W_neigh_i, ln_gamma, ln_beta, W_head, b_head, time_entity, time_item, seed_time, batch_entity, batch_item, n_id_entity, edge_index_e2i, edge_index_i2e):
    raise NotImplementedError("write your pallas kernel here")


# speedup vs baseline: 5.7938x; 5.7938x over previous
import jax
import jax.numpy as jnp
import numpy as np
from jax import lax
from jax.experimental import pallas as pl
from jax.experimental.pallas import tpu as pltpu
from jax.experimental.pallas import tpu_sc as plsc

N = 5000
NPAD = 5120
E = 160000
CH = 128
B = 1024
NUM_TILES = 16
NUM_WORKERS = 32
K = 100
NCHUNK1 = 100
NCHUNK2 = 50
RPT = NPAD // NUM_TILES
EMB_PER_W = NPAD // NUM_WORKERS
XROWS = 16384
NPAD2 = 1152
RPT2 = NPAD2 // NUM_TILES

_MESH = plsc.VectorSubcoreMesh(core_axis_name="c", subcore_axis_name="s")


def _sc_prep_body(emb_table, n_id_r, dst_r, zin, ones,
                  emb_out, cnt_out,
                  idbuf, rows, dstall, onesb, cntacc, sem):
    c = lax.axis_index("c")
    s = lax.axis_index("s")
    w = s * 2 + c
    pltpu.sync_copy(zin, cntacc.at[pl.ds(s * RPT, RPT)])
    pltpu.sync_copy(ones, onesb)
    pltpu.sync_copy(n_id_r.at[w], idbuf)
    for k in range(2):
        pltpu.async_copy(emb_table.at[idbuf.at[k]], rows, sem).wait()
        pltpu.sync_copy(rows, emb_out.at[pl.ds(w * EMB_PER_W + k * 80, 80)])
    plsc.subcore_barrier()
    pltpu.sync_copy(dst_r.at[c].at[s], dstall)

    def chunk(j, carry):
        pltpu.sync_copy(onesb, cntacc.at[dstall.at[j]], add=True)
        return carry

    lax.fori_loop(0, NCHUNK1, chunk, None)
    plsc.subcore_barrier()
    pltpu.sync_copy(cntacc.at[pl.ds(s * RPT, RPT)],
                    cnt_out.at[pl.ds(c * NPAD + s * RPT, RPT)])


_sc_prep = pl.kernel(
    _sc_prep_body,
    out_type=(
        jax.ShapeDtypeStruct((NPAD, CH), jnp.float32),
        jax.ShapeDtypeStruct((2 * NPAD, CH), jnp.float32),
    ),
    mesh=_MESH,
    scratch_types=[
        pltpu.VMEM((2, 80), jnp.int32),
        pltpu.VMEM((80, CH), jnp.float32),
        pltpu.VMEM((NCHUNK1, K), jnp.int32),
        pltpu.VMEM((K, CH), jnp.float32),
        pltpu.VMEM_SHARED((NPAD, CH), jnp.float32),
        pltpu.SemaphoreType.DMA,
    ],
)


def _make_sc_agg(nchunk, nacc, rpt):
    def body(src_r, dst_r, xcat, zin, agg_out,
             srcall, dstall, rows, acc, sem):
        c = lax.axis_index("c")
        s = lax.axis_index("s")
        pltpu.sync_copy(zin.at[pl.ds(0, rpt)], acc.at[pl.ds(s * rpt, rpt)])
        plsc.subcore_barrier()
        pltpu.sync_copy(src_r.at[c].at[s], srcall)
        pltpu.sync_copy(dst_r.at[c].at[s], dstall)

        def chunk(j, carry):
            pltpu.async_copy(xcat.at[srcall.at[j]], rows, sem).wait()
            pltpu.sync_copy(rows, acc.at[dstall.at[j]], add=True)
            return carry

        lax.fori_loop(0, nchunk, chunk, None)
        plsc.subcore_barrier()
        pltpu.sync_copy(acc.at[pl.ds(s * rpt, rpt)],
                        agg_out.at[pl.ds(c * nacc + s * rpt, rpt)])

    return pl.kernel(
        body,
        out_type=(jax.ShapeDtypeStruct((2 * nacc, CH), jnp.float32),),
        mesh=_MESH,
        scratch_types=[
            pltpu.VMEM((nchunk, K), jnp.int32),
            pltpu.VMEM((nchunk, K), jnp.int32),
            pltpu.VMEM((K, CH), jnp.float32),
            pltpu.VMEM_SHARED((nacc, CH), jnp.float32),
            pltpu.SemaphoreType.DMA,
        ],
    )


_sc_agg1 = _make_sc_agg(NCHUNK1, NPAD, RPT)
_sc_agg2 = _make_sc_agg(NCHUNK2, NPAD2, RPT2)


_LOG1E4_OVER_HALF = float(np.log(10000.0) / (CH // 2))


def _tc_encode_body(x_ref, w1_ref, b1_ref, w2_ref, b2_ref, wt_ref, bt_ref,
                    nt_ref, bat_ref, seed_ref, extra_ref, out_ref):
    x = x_ref[0]
    h = jnp.maximum(jnp.dot(x, w1_ref[0]) + b1_ref[0, 0], 0.0)
    h = jnp.dot(h, w2_ref[0]) + b2_ref[0, 0]
    iot = lax.broadcasted_iota(jnp.int32, (1, B), 1).astype(jnp.float32)
    onehot = (bat_ref[0] == iot).astype(jnp.float32)
    st = jnp.dot(onehot, seed_ref[...])
    rel = (st - nt_ref[0]) * 1e-5
    ii = lax.broadcasted_iota(jnp.int32, (1, CH // 2), 1).astype(jnp.float32)
    freqs = jnp.exp(ii * (-_LOG1E4_OVER_HALF))
    ang = rel * freqs
    pe = jnp.concatenate([jnp.sin(ang), jnp.cos(ang)], axis=1)
    out_ref[0] = (h + jnp.dot(pe, wt_ref[...]) + bt_ref[0] + extra_ref[0])


_tc_encode = pl.pallas_call(
    _tc_encode_body,
    grid=(2,),
    in_specs=[
        pl.BlockSpec((1, NPAD, CH), lambda t: (t, 0, 0)),
        pl.BlockSpec((1, CH, CH), lambda t: (t, 0, 0)),
        pl.BlockSpec((1, 1, CH), lambda t: (t, 0, 0)),
        pl.BlockSpec((1, CH, CH), lambda t: (t, 0, 0)),
        pl.BlockSpec((1, 1, CH), lambda t: (t, 0, 0)),
        pl.BlockSpec((CH, CH), lambda t: (0, 0)),
        pl.BlockSpec((1, CH), lambda t: (0, 0)),
        pl.BlockSpec((1, NPAD, 1), lambda t: (t, 0, 0)),
        pl.BlockSpec((1, NPAD, 1), lambda t: (t, 0, 0)),
        pl.BlockSpec((B, 1), lambda t: (0, 0)),
        pl.BlockSpec((1, NPAD, CH), lambda t: (t, 0, 0)),
    ],
    out_specs=pl.BlockSpec((1, NPAD, CH), lambda t: (t, 0, 0)),
    out_shape=jax.ShapeDtypeStruct((2, NPAD, CH), jnp.float32),
)


def _tc_layer_body(x_ref, agg_ref, cnt_ref, ws_ref, wn_ref, out_ref):
    mean = agg_ref[0] / jnp.maximum(cnt_ref[0], 1.0)
    out_ref[0] = jnp.maximum(
        jnp.dot(x_ref[0], ws_ref[0]) + jnp.dot(mean, wn_ref[0]), 0.0)


_tc_layer = pl.pallas_call(
    _tc_layer_body,
    grid=(2,),
    in_specs=[
        pl.BlockSpec((1, NPAD, CH), lambda t: (t, 0, 0)),
        pl.BlockSpec((1, NPAD, CH), lambda t: (t, 0, 0)),
        pl.BlockSpec((1, NPAD, CH), lambda t: (t, 0, 0)),
        pl.BlockSpec((1, CH, CH), lambda t: (t, 0, 0)),
        pl.BlockSpec((1, CH, CH), lambda t: (t, 0, 0)),
    ],
    out_specs=pl.BlockSpec((1, NPAD, CH), lambda t: (t, 0, 0)),
    out_shape=jax.ShapeDtypeStruct((2, NPAD, CH), jnp.float32),
)


def _tc_head_body(x_ref, p0_ref, p1_ref, cnt_ref, ws_ref, wn_ref,
                  g_ref, bt_ref, wh_ref, bh_ref, out_ref):
    mean = (p0_ref[...] + p1_ref[...]) / jnp.maximum(cnt_ref[...], 1.0)
    h = jnp.maximum(
        jnp.dot(x_ref[...], ws_ref[...]) + jnp.dot(mean, wn_ref[...]), 0.0)
    mu = jnp.mean(h, axis=-1, keepdims=True)
    var = jnp.mean((h - mu) ** 2, axis=-1, keepdims=True)
    hn = (h - mu) * lax.rsqrt(var + 1e-5) * g_ref[0] + bt_ref[0]
    out_ref[...] = jnp.dot(hn, wh_ref[...]) + bh_ref[0]


_tc_head = pl.pallas_call(
    _tc_head_body,
    out_shape=jax.ShapeDtypeStruct((B, CH), jnp.float32),
)


def _pad_rows(a, n):
    return jnp.pad(a, ((0, n - a.shape[0]),) + ((0, 0),) * (a.ndim - 1))


def kernel(x_entity, x_item, W_fe_e, b_fe_e, W_fe_i, b_fe_i, W_ne_e, b_ne_e,
           W_ne_i, b_ne_i, W_time, b_time, emb_table, W_self_e, W_neigh_e,
           W_self_i, W_neigh_i, ln_gamma, ln_beta, W_head, b_head,
           time_entity, time_item, seed_time, batch_entity, batch_item,
           n_id_entity, edge_index_e2i, edge_index_i2e):
    src1 = jnp.stack([
        (edge_index_i2e[0] + NPAD).reshape(NUM_TILES, NCHUNK1, K),
        edge_index_e2i[0].reshape(NUM_TILES, NCHUNK1, K)])
    dst1 = jnp.stack([
        edge_index_i2e[1].reshape(NUM_TILES, NCHUNK1, K),
        edge_index_e2i[1].reshape(NUM_TILES, NCHUNK1, K)])
    src2 = (edge_index_i2e[0] + NPAD).reshape(2, NUM_TILES, NCHUNK2, K)
    dst2 = jnp.where(edge_index_i2e[1] < B, edge_index_i2e[1],
                     B).reshape(2, NUM_TILES, NCHUNK2, K)

    n_id_r = _pad_rows(n_id_entity, NPAD).reshape(NUM_WORKERS, 2, 80)
    zeros_t = jnp.zeros((RPT, CH), jnp.float32)
    ones_k = jnp.ones((K, CH), jnp.float32)

    emb_rows, cnt_flat = _sc_prep(emb_table, n_id_r, dst1, zeros_t, ones_k)
    cnt = cnt_flat.reshape(2, NPAD, CH)

    x_all = jnp.stack([_pad_rows(x_entity, NPAD), _pad_rows(x_item, NPAD)])
    w1 = jnp.stack([W_fe_e, W_fe_i])
    b1 = jnp.stack([b_fe_e, b_fe_i]).reshape(2, 1, CH)
    w2 = jnp.stack([W_ne_e, W_ne_i])
    b2 = jnp.stack([b_ne_e, b_ne_i]).reshape(2, 1, CH)
    nt = jnp.stack([_pad_rows(time_entity, NPAD), _pad_rows(time_item, NPAD)])
    nt = nt.astype(jnp.float32).reshape(2, NPAD, 1)
    bat = jnp.stack([_pad_rows(batch_entity, NPAD),
                     _pad_rows(batch_item, NPAD)])
    bat = bat.astype(jnp.float32).reshape(2, NPAD, 1)
    seed_col = seed_time.astype(jnp.float32).reshape(B, 1)
    extra = jnp.stack([emb_rows, jnp.zeros_like(emb_rows)])

    x0 = _tc_encode(x_all, w1, b1, w2, b2, W_time, b_time.reshape(1, CH),
                    nt, bat, seed_col, extra)

    xcat0 = _pad_rows(x0.reshape(2 * NPAD, CH), XROWS)
    (agg,) = _sc_agg1(src1, dst1, xcat0, zeros_t)
    agg = agg.reshape(2, NPAD, CH)
    ws1 = jnp.stack([W_self_e[0], W_self_i[0]])
    wn1 = jnp.stack([W_neigh_e[0], W_neigh_i[0]])
    x1 = _tc_layer(x0, agg, cnt, ws1, wn1)

    xcat1 = _pad_rows(x1.reshape(2 * NPAD, CH), XROWS)
    (p,) = _sc_agg2(src2, dst2, xcat1, zeros_t)
    out = _tc_head(x1[0, :B], p[:B], p[NPAD2:NPAD2 + B], cnt[0, :B],
                   W_self_e[1], W_neigh_e[1],
                   ln_gamma.reshape(1, CH), ln_beta.reshape(1, CH),
                   W_head, b_head.reshape(1, CH))
    return out


# speedup vs baseline: 7.1571x; 1.2353x over previous
import jax
import jax.numpy as jnp
import numpy as np
from jax import lax
from jax.experimental import pallas as pl
from jax.experimental.pallas import tpu as pltpu
from jax.experimental.pallas import tpu_sc as plsc

N = 5000
NPAD = 5120
E = 160000
CH = 128
B = 1024
NUM_TILES = 16
NUM_WORKERS = 32
K = 100
NCHUNK1 = 100
NCHUNK2 = 50
RPT = NPAD // NUM_TILES
EMB_PER_W = NPAD // NUM_WORKERS
XROWS = 16384
NPAD2 = 1152
RPT2 = NPAD2 // NUM_TILES

_MESH = plsc.VectorSubcoreMesh(core_axis_name="c", subcore_axis_name="s")


def _sc_prep_body(emb_table, n_id_r, dst_r, zin, ones,
                  emb_out, cnt_out,
                  idbuf, rows, dstall, onesb, cntacc, sem):
    c = lax.axis_index("c")
    s = lax.axis_index("s")
    w = s * 2 + c
    pltpu.sync_copy(zin, cntacc.at[pl.ds(s * RPT, RPT)])
    pltpu.sync_copy(ones, onesb)
    pltpu.sync_copy(n_id_r.at[w], idbuf)
    for k in range(2):
        pltpu.async_copy(emb_table.at[idbuf.at[k]], rows, sem).wait()
        pltpu.sync_copy(rows, emb_out.at[pl.ds(w * EMB_PER_W + k * 80, 80)])
    plsc.subcore_barrier()
    pltpu.sync_copy(dst_r.at[c].at[s], dstall)

    def chunk(j, carry):
        pltpu.sync_copy(onesb, cntacc.at[dstall.at[j]], add=True)
        return carry

    lax.fori_loop(0, NCHUNK1, chunk, None)
    plsc.subcore_barrier()
    pltpu.sync_copy(cntacc.at[pl.ds(s * RPT, RPT)],
                    cnt_out.at[pl.ds(c * NPAD + s * RPT, RPT)])


_sc_prep = pl.kernel(
    _sc_prep_body,
    out_type=(
        jax.ShapeDtypeStruct((NPAD, CH), jnp.float32),
        jax.ShapeDtypeStruct((2 * NPAD, CH), jnp.float32),
    ),
    mesh=_MESH,
    scratch_types=[
        pltpu.VMEM((2, 80), jnp.int32),
        pltpu.VMEM((80, CH), jnp.float32),
        pltpu.VMEM((NCHUNK1, K), jnp.int32),
        pltpu.VMEM((K, CH), jnp.float32),
        pltpu.VMEM_SHARED((NPAD, CH), jnp.float32),
        pltpu.SemaphoreType.DMA,
    ],
)


def _make_sc_agg(nchunk, nacc, rpt):
    def body(src_r, dst_r, xcat, zin, agg_out,
             srcall, dstall, rows0, rows1, acc, sem0, sem1):
        c = lax.axis_index("c")
        s = lax.axis_index("s")
        pltpu.sync_copy(zin.at[pl.ds(0, rpt)], acc.at[pl.ds(s * rpt, rpt)])
        plsc.subcore_barrier()
        pltpu.sync_copy(src_r.at[c].at[s], srcall)
        pltpu.sync_copy(dst_r.at[c].at[s], dstall)

        bufs = ((rows0, sem0), (rows1, sem1))

        def gstart(j, b):
            rows, sem = bufs[b]
            pltpu.async_copy(xcat.at[srcall.at[j]], rows, sem)

        def gdone(j, b):
            rows, sem = bufs[b]
            pltpu.make_async_copy(xcat.at[srcall.at[j]], rows, sem).wait()
            pltpu.sync_copy(rows, acc.at[dstall.at[j]], add=True)

        gstart(0, 0)
        gstart(1, 1)

        def chunk(jj, carry):
            j = 2 * jj
            gdone(j, 0)
            gstart(j + 2, 0)
            gdone(j + 1, 1)
            gstart(j + 3, 1)
            return carry

        lax.fori_loop(0, nchunk // 2 - 1, chunk, None)
        gdone(nchunk - 2, 0)
        gdone(nchunk - 1, 1)
        plsc.subcore_barrier()
        pltpu.sync_copy(acc.at[pl.ds(s * rpt, rpt)],
                        agg_out.at[pl.ds(c * nacc + s * rpt, rpt)])

    return pl.kernel(
        body,
        out_type=(jax.ShapeDtypeStruct((2 * nacc, CH), jnp.float32),),
        mesh=_MESH,
        scratch_types=[
            pltpu.VMEM((nchunk, K), jnp.int32),
            pltpu.VMEM((nchunk, K), jnp.int32),
            pltpu.VMEM((K, CH), jnp.float32),
            pltpu.VMEM((K, CH), jnp.float32),
            pltpu.VMEM_SHARED((nacc, CH), jnp.float32),
            pltpu.SemaphoreType.DMA,
            pltpu.SemaphoreType.DMA,
        ],
    )


_sc_agg1 = _make_sc_agg(NCHUNK1, NPAD, RPT)
_sc_agg2 = _make_sc_agg(NCHUNK2, NPAD2, RPT2)


_LOG1E4_OVER_HALF = float(np.log(10000.0) / (CH // 2))


def _tc_encode_body(x_ref, w1_ref, b1_ref, w2_ref, b2_ref, wt_ref, bt_ref,
                    nt_ref, bat_ref, seed_ref, extra_ref, out_ref):
    x = x_ref[0]
    h = jnp.maximum(jnp.dot(x, w1_ref[0]) + b1_ref[0, 0], 0.0)
    h = jnp.dot(h, w2_ref[0]) + b2_ref[0, 0]
    iot = lax.broadcasted_iota(jnp.int32, (1, B), 1).astype(jnp.float32)
    onehot = (bat_ref[0] == iot).astype(jnp.float32)
    st = jnp.dot(onehot, seed_ref[...])
    rel = (st - nt_ref[0]) * 1e-5
    ii = lax.broadcasted_iota(jnp.int32, (1, CH // 2), 1).astype(jnp.float32)
    freqs = jnp.exp(ii * (-_LOG1E4_OVER_HALF))
    ang = rel * freqs
    pe = jnp.concatenate([jnp.sin(ang), jnp.cos(ang)], axis=1)
    out_ref[0] = (h + jnp.dot(pe, wt_ref[...]) + bt_ref[0] + extra_ref[0])


_tc_encode = pl.pallas_call(
    _tc_encode_body,
    grid=(2,),
    in_specs=[
        pl.BlockSpec((1, NPAD, CH), lambda t: (t, 0, 0)),
        pl.BlockSpec((1, CH, CH), lambda t: (t, 0, 0)),
        pl.BlockSpec((1, 1, CH), lambda t: (t, 0, 0)),
        pl.BlockSpec((1, CH, CH), lambda t: (t, 0, 0)),
        pl.BlockSpec((1, 1, CH), lambda t: (t, 0, 0)),
        pl.BlockSpec((CH, CH), lambda t: (0, 0)),
        pl.BlockSpec((1, CH), lambda t: (0, 0)),
        pl.BlockSpec((1, NPAD, 1), lambda t: (t, 0, 0)),
        pl.BlockSpec((1, NPAD, 1), lambda t: (t, 0, 0)),
        pl.BlockSpec((B, 1), lambda t: (0, 0)),
        pl.BlockSpec((1, NPAD, CH), lambda t: (t, 0, 0)),
    ],
    out_specs=pl.BlockSpec((1, NPAD, CH), lambda t: (t, 0, 0)),
    out_shape=jax.ShapeDtypeStruct((2, NPAD, CH), jnp.float32),
)


def _tc_layer_body(x_ref, agg_ref, cnt_ref, ws_ref, wn_ref, out_ref):
    mean = agg_ref[0] / jnp.maximum(cnt_ref[0], 1.0)
    out_ref[0] = jnp.maximum(
        jnp.dot(x_ref[0], ws_ref[0]) + jnp.dot(mean, wn_ref[0]), 0.0)


_tc_layer = pl.pallas_call(
    _tc_layer_body,
    grid=(2,),
    in_specs=[
        pl.BlockSpec((1, NPAD, CH), lambda t: (t, 0, 0)),
        pl.BlockSpec((1, NPAD, CH), lambda t: (t, 0, 0)),
        pl.BlockSpec((1, NPAD, CH), lambda t: (t, 0, 0)),
        pl.BlockSpec((1, CH, CH), lambda t: (t, 0, 0)),
        pl.BlockSpec((1, CH, CH), lambda t: (t, 0, 0)),
    ],
    out_specs=pl.BlockSpec((1, NPAD, CH), lambda t: (t, 0, 0)),
    out_shape=jax.ShapeDtypeStruct((2, NPAD, CH), jnp.float32),
)


def _tc_head_body(x_ref, p0_ref, p1_ref, cnt_ref, ws_ref, wn_ref,
                  g_ref, bt_ref, wh_ref, bh_ref, out_ref):
    mean = (p0_ref[...] + p1_ref[...]) / jnp.maximum(cnt_ref[...], 1.0)
    h = jnp.maximum(
        jnp.dot(x_ref[...], ws_ref[...]) + jnp.dot(mean, wn_ref[...]), 0.0)
    mu = jnp.mean(h, axis=-1, keepdims=True)
    var = jnp.mean((h - mu) ** 2, axis=-1, keepdims=True)
    hn = (h - mu) * lax.rsqrt(var + 1e-5) * g_ref[0] + bt_ref[0]
    out_ref[...] = jnp.dot(hn, wh_ref[...]) + bh_ref[0]


_tc_head = pl.pallas_call(
    _tc_head_body,
    out_shape=jax.ShapeDtypeStruct((B, CH), jnp.float32),
)


def _pad_rows(a, n):
    return jnp.pad(a, ((0, n - a.shape[0]),) + ((0, 0),) * (a.ndim - 1))


def kernel(x_entity, x_item, W_fe_e, b_fe_e, W_fe_i, b_fe_i, W_ne_e, b_ne_e,
           W_ne_i, b_ne_i, W_time, b_time, emb_table, W_self_e, W_neigh_e,
           W_self_i, W_neigh_i, ln_gamma, ln_beta, W_head, b_head,
           time_entity, time_item, seed_time, batch_entity, batch_item,
           n_id_entity, edge_index_e2i, edge_index_i2e):
    src1 = jnp.stack([
        (edge_index_i2e[0] + NPAD).reshape(NUM_TILES, NCHUNK1, K),
        edge_index_e2i[0].reshape(NUM_TILES, NCHUNK1, K)])
    dst1 = jnp.stack([
        edge_index_i2e[1].reshape(NUM_TILES, NCHUNK1, K),
        edge_index_e2i[1].reshape(NUM_TILES, NCHUNK1, K)])
    src2 = (edge_index_i2e[0] + NPAD).reshape(2, NUM_TILES, NCHUNK2, K)
    dst2 = jnp.where(edge_index_i2e[1] < B, edge_index_i2e[1],
                     B).reshape(2, NUM_TILES, NCHUNK2, K)

    n_id_r = _pad_rows(n_id_entity, NPAD).reshape(NUM_WORKERS, 2, 80)
    zeros_t = jnp.zeros((RPT, CH), jnp.float32)
    ones_k = jnp.ones((K, CH), jnp.float32)

    emb_rows, cnt_flat = _sc_prep(emb_table, n_id_r, dst1, zeros_t, ones_k)
    cnt = cnt_flat.reshape(2, NPAD, CH)

    x_all = jnp.stack([_pad_rows(x_entity, NPAD), _pad_rows(x_item, NPAD)])
    w1 = jnp.stack([W_fe_e, W_fe_i])
    b1 = jnp.stack([b_fe_e, b_fe_i]).reshape(2, 1, CH)
    w2 = jnp.stack([W_ne_e, W_ne_i])
    b2 = jnp.stack([b_ne_e, b_ne_i]).reshape(2, 1, CH)
    nt = jnp.stack([_pad_rows(time_entity, NPAD), _pad_rows(time_item, NPAD)])
    nt = nt.astype(jnp.float32).reshape(2, NPAD, 1)
    bat = jnp.stack([_pad_rows(batch_entity, NPAD),
                     _pad_rows(batch_item, NPAD)])
    bat = bat.astype(jnp.float32).reshape(2, NPAD, 1)
    seed_col = seed_time.astype(jnp.float32).reshape(B, 1)
    extra = jnp.stack([emb_rows, jnp.zeros_like(emb_rows)])

    x0 = _tc_encode(x_all, w1, b1, w2, b2, W_time, b_time.reshape(1, CH),
                    nt, bat, seed_col, extra)

    xcat0 = _pad_rows(x0.reshape(2 * NPAD, CH), XROWS)
    (agg,) = _sc_agg1(src1, dst1, xcat0, zeros_t)
    agg = agg.reshape(2, NPAD, CH)
    ws1 = jnp.stack([W_self_e[0], W_self_i[0]])
    wn1 = jnp.stack([W_neigh_e[0], W_neigh_i[0]])
    x1 = _tc_layer(x0, agg, cnt, ws1, wn1)

    xcat1 = _pad_rows(x1.reshape(2 * NPAD, CH), XROWS)
    (p,) = _sc_agg2(src2, dst2, xcat1, zeros_t)
    out = _tc_head(x1[0, :B], p[:B], p[NPAD2:NPAD2 + B], cnt[0, :B],
                   W_self_e[1], W_neigh_e[1],
                   ln_gamma.reshape(1, CH), ln_beta.reshape(1, CH),
                   W_head, b_head.reshape(1, CH))
    return out


# speedup vs baseline: 8.2055x; 1.1465x over previous
import jax
import jax.numpy as jnp
import numpy as np
from jax import lax
from jax.experimental import pallas as pl
from jax.experimental.pallas import tpu as pltpu
from jax.experimental.pallas import tpu_sc as plsc

N = 5000
NPAD = 5120
E = 160000
CH = 128
B = 1024
NUM_TILES = 16
NUM_WORKERS = 32
K = 100
NCHUNK1 = 100
NCHUNK2 = 50
RPT = NPAD // NUM_TILES
EMB_PER_W = NPAD // NUM_WORKERS
XROWS = 16384
NPAD2 = 1152
RPT2 = NPAD2 // NUM_TILES

_MESH = plsc.VectorSubcoreMesh(core_axis_name="c", subcore_axis_name="s")


def _sc_emb_body(emb_table, n_id_r, emb_out, idbuf, rows, sem):
    c = lax.axis_index("c")
    s = lax.axis_index("s")
    w = s * 2 + c
    pltpu.sync_copy(n_id_r.at[w], idbuf)
    for k in range(2):
        pltpu.async_copy(emb_table.at[idbuf.at[k]], rows, sem).wait()
        pltpu.sync_copy(rows, emb_out.at[pl.ds(w * EMB_PER_W + k * 80, 80)])


_sc_emb = pl.kernel(
    _sc_emb_body,
    out_type=jax.ShapeDtypeStruct((NPAD, CH), jnp.float32),
    mesh=_MESH,
    scratch_types=[
        pltpu.VMEM((2, 80), jnp.int32),
        pltpu.VMEM((80, CH), jnp.float32),
        pltpu.SemaphoreType.DMA,
    ],
)


def _sc_cnt_body(dst_r, zin, ones, cnt_out, dstall, onesb, cntacc, sem):
    c = lax.axis_index("c")
    s = lax.axis_index("s")
    pltpu.sync_copy(zin, cntacc.at[pl.ds(s * RPT, RPT)])
    pltpu.sync_copy(ones, onesb)
    plsc.subcore_barrier()
    pltpu.sync_copy(dst_r.at[c].at[s], dstall)

    def cstart(j):
        pltpu.async_copy(onesb, cntacc.at[dstall.at[j]], sem, add=True)

    def cwait(j):
        pltpu.make_async_copy(onesb, cntacc.at[dstall.at[j]], sem).wait()

    cstart(0)
    cstart(1)

    def chunk(j, carry):
        cwait(j)
        cstart(j + 2)
        return carry

    lax.fori_loop(0, NCHUNK1 - 2, chunk, None)
    cwait(NCHUNK1 - 2)
    cwait(NCHUNK1 - 1)
    plsc.subcore_barrier()
    pltpu.sync_copy(cntacc.at[pl.ds(s * RPT, RPT)],
                    cnt_out.at[pl.ds(c * NPAD + s * RPT, RPT)])


_sc_cnt = pl.kernel(
    _sc_cnt_body,
    out_type=jax.ShapeDtypeStruct((2 * NPAD, CH), jnp.float32),
    mesh=_MESH,
    scratch_types=[
        pltpu.VMEM((NCHUNK1, K), jnp.int32),
        pltpu.VMEM((K, CH), jnp.float32),
        pltpu.VMEM_SHARED((NPAD, CH), jnp.float32),
        pltpu.SemaphoreType.DMA,
    ],
)


def _make_sc_agg(nchunk, nacc, rpt):
    def body(src_r, dst_r, xcat, zin, agg_out,
             srcall, dstall, rows0, rows1, acc, sem0, sem1):
        c = lax.axis_index("c")
        s = lax.axis_index("s")
        pltpu.sync_copy(zin.at[pl.ds(0, rpt)], acc.at[pl.ds(s * rpt, rpt)])
        plsc.subcore_barrier()
        pltpu.sync_copy(src_r.at[c].at[s], srcall)
        pltpu.sync_copy(dst_r.at[c].at[s], dstall)

        bufs = ((rows0, sem0), (rows1, sem1))

        def gstart(j, b):
            rows, sem = bufs[b]
            pltpu.async_copy(xcat.at[srcall.at[j]], rows, sem)

        def gdone(j, b):
            rows, sem = bufs[b]
            pltpu.make_async_copy(xcat.at[srcall.at[j]], rows, sem).wait()
            pltpu.sync_copy(rows, acc.at[dstall.at[j]], add=True)

        gstart(0, 0)
        gstart(1, 1)

        def chunk(jj, carry):
            j = 2 * jj
            gdone(j, 0)
            gstart(j + 2, 0)
            gdone(j + 1, 1)
            gstart(j + 3, 1)
            return carry

        lax.fori_loop(0, nchunk // 2 - 1, chunk, None)
        gdone(nchunk - 2, 0)
        gdone(nchunk - 1, 1)
        plsc.subcore_barrier()
        pltpu.sync_copy(acc.at[pl.ds(s * rpt, rpt)],
                        agg_out.at[pl.ds(c * nacc + s * rpt, rpt)])

    return pl.kernel(
        body,
        out_type=(jax.ShapeDtypeStruct((2 * nacc, CH), jnp.float32),),
        mesh=_MESH,
        scratch_types=[
            pltpu.VMEM((nchunk, K), jnp.int32),
            pltpu.VMEM((nchunk, K), jnp.int32),
            pltpu.VMEM((K, CH), jnp.float32),
            pltpu.VMEM((K, CH), jnp.float32),
            pltpu.VMEM_SHARED((nacc, CH), jnp.float32),
            pltpu.SemaphoreType.DMA,
            pltpu.SemaphoreType.DMA,
        ],
    )


_sc_agg1 = _make_sc_agg(NCHUNK1, NPAD, RPT)
_sc_agg2 = _make_sc_agg(NCHUNK2, NPAD2, RPT2)


_LOG1E4_OVER_HALF = float(np.log(10000.0) / (CH // 2))


def _tc_encode_body(x_ref, w1_ref, b1_ref, w2_ref, b2_ref, wt_ref, bt_ref,
                    nt_ref, bat_ref, seed_ref, extra_ref, out_ref):
    x = x_ref[0]
    h = jnp.maximum(jnp.dot(x, w1_ref[0]) + b1_ref[0, 0], 0.0)
    h = jnp.dot(h, w2_ref[0]) + b2_ref[0, 0]
    iot = lax.broadcasted_iota(jnp.int32, (1, B), 1).astype(jnp.float32)
    onehot = (bat_ref[0] == iot).astype(jnp.float32)
    st = jnp.dot(onehot, seed_ref[...])
    rel = (st - nt_ref[0]) * 1e-5
    ii = lax.broadcasted_iota(jnp.int32, (1, CH // 2), 1).astype(jnp.float32)
    freqs = jnp.exp(ii * (-_LOG1E4_OVER_HALF))
    ang = rel * freqs
    pe = jnp.concatenate([jnp.sin(ang), jnp.cos(ang)], axis=1)
    out_ref[0] = (h + jnp.dot(pe, wt_ref[...]) + bt_ref[0] + extra_ref[0])


_tc_encode = pl.pallas_call(
    _tc_encode_body,
    grid=(2,),
    in_specs=[
        pl.BlockSpec((1, NPAD, CH), lambda t: (t, 0, 0)),
        pl.BlockSpec((1, CH, CH), lambda t: (t, 0, 0)),
        pl.BlockSpec((1, 1, CH), lambda t: (t, 0, 0)),
        pl.BlockSpec((1, CH, CH), lambda t: (t, 0, 0)),
        pl.BlockSpec((1, 1, CH), lambda t: (t, 0, 0)),
        pl.BlockSpec((CH, CH), lambda t: (0, 0)),
        pl.BlockSpec((1, CH), lambda t: (0, 0)),
        pl.BlockSpec((1, NPAD, 1), lambda t: (t, 0, 0)),
        pl.BlockSpec((1, NPAD, 1), lambda t: (t, 0, 0)),
        pl.BlockSpec((B, 1), lambda t: (0, 0)),
        pl.BlockSpec((1, NPAD, CH), lambda t: (t, 0, 0)),
    ],
    out_specs=pl.BlockSpec((1, NPAD, CH), lambda t: (t, 0, 0)),
    out_shape=jax.ShapeDtypeStruct((2, NPAD, CH), jnp.float32),
)


def _tc_layer_body(x_ref, agg_ref, cnt_ref, ws_ref, wn_ref, out_ref):
    mean = agg_ref[0] / jnp.maximum(cnt_ref[0], 1.0)
    out_ref[0] = jnp.maximum(
        jnp.dot(x_ref[0], ws_ref[0]) + jnp.dot(mean, wn_ref[0]), 0.0)


_tc_layer = pl.pallas_call(
    _tc_layer_body,
    grid=(2,),
    in_specs=[
        pl.BlockSpec((1, NPAD, CH), lambda t: (t, 0, 0)),
        pl.BlockSpec((1, NPAD, CH), lambda t: (t, 0, 0)),
        pl.BlockSpec((1, NPAD, CH), lambda t: (t, 0, 0)),
        pl.BlockSpec((1, CH, CH), lambda t: (t, 0, 0)),
        pl.BlockSpec((1, CH, CH), lambda t: (t, 0, 0)),
    ],
    out_specs=pl.BlockSpec((1, NPAD, CH), lambda t: (t, 0, 0)),
    out_shape=jax.ShapeDtypeStruct((2, NPAD, CH), jnp.float32),
)


def _tc_head_body(x_ref, p0_ref, p1_ref, cnt_ref, ws_ref, wn_ref,
                  g_ref, bt_ref, wh_ref, bh_ref, out_ref):
    mean = (p0_ref[...] + p1_ref[...]) / jnp.maximum(cnt_ref[...], 1.0)
    h = jnp.maximum(
        jnp.dot(x_ref[...], ws_ref[...]) + jnp.dot(mean, wn_ref[...]), 0.0)
    mu = jnp.mean(h, axis=-1, keepdims=True)
    var = jnp.mean((h - mu) ** 2, axis=-1, keepdims=True)
    hn = (h - mu) * lax.rsqrt(var + 1e-5) * g_ref[0] + bt_ref[0]
    out_ref[...] = jnp.dot(hn, wh_ref[...]) + bh_ref[0]


_tc_head = pl.pallas_call(
    _tc_head_body,
    out_shape=jax.ShapeDtypeStruct((B, CH), jnp.float32),
)


def _pad_rows(a, n):
    return jnp.pad(a, ((0, n - a.shape[0]),) + ((0, 0),) * (a.ndim - 1))


def kernel(x_entity, x_item, W_fe_e, b_fe_e, W_fe_i, b_fe_i, W_ne_e, b_ne_e,
           W_ne_i, b_ne_i, W_time, b_time, emb_table, W_self_e, W_neigh_e,
           W_self_i, W_neigh_i, ln_gamma, ln_beta, W_head, b_head,
           time_entity, time_item, seed_time, batch_entity, batch_item,
           n_id_entity, edge_index_e2i, edge_index_i2e):
    src1 = jnp.stack([
        (edge_index_i2e[0] + NPAD).reshape(NUM_TILES, NCHUNK1, K),
        edge_index_e2i[0].reshape(NUM_TILES, NCHUNK1, K)])
    dst1 = jnp.stack([
        edge_index_i2e[1].reshape(NUM_TILES, NCHUNK1, K),
        edge_index_e2i[1].reshape(NUM_TILES, NCHUNK1, K)])
    src2 = (edge_index_i2e[0] + NPAD).reshape(2, NUM_TILES, NCHUNK2, K)
    dst2 = jnp.where(edge_index_i2e[1] < B, edge_index_i2e[1],
                     B).reshape(2, NUM_TILES, NCHUNK2, K)

    n_id_r = _pad_rows(n_id_entity, NPAD).reshape(NUM_WORKERS, 2, 80)
    zeros_t = jnp.zeros((RPT, CH), jnp.float32)
    ones_k = jnp.ones((K, CH), jnp.float32)

    cnt = _sc_cnt(dst1, zeros_t, ones_k).reshape(2, NPAD, CH)
    emb_rows = _sc_emb(emb_table, n_id_r)

    x_all = jnp.stack([_pad_rows(x_entity, NPAD), _pad_rows(x_item, NPAD)])
    w1 = jnp.stack([W_fe_e, W_fe_i])
    b1 = jnp.stack([b_fe_e, b_fe_i]).reshape(2, 1, CH)
    w2 = jnp.stack([W_ne_e, W_ne_i])
    b2 = jnp.stack([b_ne_e, b_ne_i]).reshape(2, 1, CH)
    nt = jnp.stack([_pad_rows(time_entity, NPAD), _pad_rows(time_item, NPAD)])
    nt = nt.astype(jnp.float32).reshape(2, NPAD, 1)
    bat = jnp.stack([_pad_rows(batch_entity, NPAD),
                     _pad_rows(batch_item, NPAD)])
    bat = bat.astype(jnp.float32).reshape(2, NPAD, 1)
    seed_col = seed_time.astype(jnp.float32).reshape(B, 1)
    extra = jnp.stack([emb_rows, jnp.zeros_like(emb_rows)])

    x0 = _tc_encode(x_all, w1, b1, w2, b2, W_time, b_time.reshape(1, CH),
                    nt, bat, seed_col, extra)

    xcat0 = _pad_rows(x0.reshape(2 * NPAD, CH), XROWS)
    (agg,) = _sc_agg1(src1, dst1, xcat0, zeros_t)
    agg = agg.reshape(2, NPAD, CH)
    ws1 = jnp.stack([W_self_e[0], W_self_i[0]])
    wn1 = jnp.stack([W_neigh_e[0], W_neigh_i[0]])
    x1 = _tc_layer(x0, agg, cnt, ws1, wn1)

    xcat1 = _pad_rows(x1.reshape(2 * NPAD, CH), XROWS)
    (p,) = _sc_agg2(src2, dst2, xcat1, zeros_t)
    out = _tc_head(x1[0, :B], p[:B], p[NPAD2:NPAD2 + B], cnt[0, :B],
                   W_self_e[1], W_neigh_e[1],
                   ln_gamma.reshape(1, CH), ln_beta.reshape(1, CH),
                   W_head, b_head.reshape(1, CH))
    return out
